# baseline (speedup 1.0000x reference)
import functools

import jax
import jax.numpy as jnp
from jax import lax
from jax.experimental import pallas as pl
from jax.experimental.pallas import tpu as pltpu
from jax.experimental.pallas import tpu_sc as plsc

INPUT_DIM = 1024
LATENT_DIM = 64
NUM_EMBEDDINGS = 1024
BATCH = 16384

TILE = 512
NB = BATCH // TILE

_SC_INFO = plsc.get_sparse_core_info()
NW = _SC_INFO.num_cores * _SC_INFO.num_subcores
B_PER_W = BATCH // NW
CHUNK = 64


def _idx_body(x_ref, we_ref, be_ref, emb_ref, idx_ref):
    x = x_ref[...]
    z = jnp.dot(x, we_ref[...], preferred_element_type=jnp.float32) + be_ref[...]
    sim = jnp.dot(z, emb_ref[...], preferred_element_type=jnp.float32)
    e2 = jnp.sum(emb_ref[...] ** 2, axis=0, keepdims=True)
    d = jnp.sum(z * z, axis=1, keepdims=True) + e2 - 2.0 * sim
    idx_ref[0] = jnp.argmin(d, axis=1)[None, :]


def _table_body(embt_ref, wd_ref, bd_ref, cb_ref):
    cb_ref[...] = (jnp.dot(embt_ref[...], wd_ref[...],
                           preferred_element_type=jnp.float32) + bd_ref[...])


def _make_sc_gather():
    mesh = plsc.VectorSubcoreMesh(core_axis_name="c", subcore_axis_name="s")

    @functools.partial(
        pl.kernel, mesh=mesh,
        out_type=jax.ShapeDtypeStruct((BATCH, INPUT_DIM), jnp.float32),
        scratch_types=[
            pltpu.VMEM((B_PER_W,), jnp.int32),
            pltpu.VMEM((CHUNK, INPUT_DIM), jnp.float32),
            pltpu.SemaphoreType.DMA,
        ],
    )
    def sc_gather(cb_hbm, idx_hbm, out_hbm, idx_v, rows_v, sem):
        wid = lax.axis_index("s") * _SC_INFO.num_cores + lax.axis_index("c")
        base = wid * B_PER_W
        pltpu.sync_copy(idx_hbm.at[pl.ds(base, B_PER_W)], idx_v)
        for c in range(B_PER_W // CHUNK):
            pltpu.async_copy(
                cb_hbm.at[idx_v.at[pl.ds(c * CHUNK, CHUNK)]], rows_v, sem
            ).wait()
            pltpu.sync_copy(rows_v, out_hbm.at[pl.ds(base + c * CHUNK, CHUNK)])

    return sc_gather


_sc_gather = _make_sc_gather()


@jax.jit
def kernel(x, W_enc, b_enc, W_emb, W_dec, b_dec):
    full = lambda shape: pl.BlockSpec(shape, lambda i: (0,) * len(shape))
    idx3 = pl.pallas_call(
        _idx_body,
        grid=(NB,),
        in_specs=[
            pl.BlockSpec((TILE, INPUT_DIM), lambda i: (i, 0)),
            full((INPUT_DIM, LATENT_DIM)),
            full((1, LATENT_DIM)),
            full((LATENT_DIM, NUM_EMBEDDINGS)),
        ],
        out_specs=pl.BlockSpec((1, 1, TILE), lambda i: (i, 0, 0)),
        out_shape=jax.ShapeDtypeStruct((NB, 1, TILE), jnp.int32),
    )(x, W_enc, b_enc.reshape(1, -1), W_emb)
    idx = idx3.reshape(BATCH)

    cb = pl.pallas_call(
        _table_body,
        in_specs=[
            pl.BlockSpec((NUM_EMBEDDINGS, LATENT_DIM), lambda: (0, 0)),
            pl.BlockSpec((LATENT_DIM, INPUT_DIM), lambda: (0, 0)),
            pl.BlockSpec((1, INPUT_DIM), lambda: (0, 0)),
        ],
        out_specs=pl.BlockSpec((NUM_EMBEDDINGS, INPUT_DIM), lambda: (0, 0)),
        out_shape=jax.ShapeDtypeStruct((NUM_EMBEDDINGS, INPUT_DIM), jnp.float32),
    )(W_emb.T, W_dec, b_dec.reshape(1, -1))

    return _sc_gather(cb, idx)
